# Initial kernel scaffold; baseline (speedup 1.0000x reference)
import jax
import jax.numpy as jnp
from jax.experimental import pallas as pl


def kernel(confidences, hits, labels):
    raise NotImplementedError("write your pallas kernel here")


# speedup vs baseline: 65.8242x; 65.8242x over previous
import jax
import jax.numpy as jnp
from jax import lax
from jax.experimental import pallas as pl
from jax.experimental.pallas import tpu as pltpu
from jax.experimental.pallas import tpu_sc as plsc

NUM_CLASS = 100
N_BINS = 15
PAD_CLASS = 100
C_PAD = 128
NW = 32
LANES = 16
SUB = 4096
TILE = 512
N_PADDED = NW * LANES * SUB
ROWS = NW * LANES
ACC_W = 1504
HIST_N = ROWS * C_PAD
ACC_N = ROWS * ACC_W


def _histogram_body(lab_hbm, hist_hbm, lab_t, hist_v):
    w = lax.axis_index("s") * 2 + lax.axis_index("c")
    lane = lax.iota(jnp.int32, LANES)
    lane_c = lane * C_PAD
    ones = jnp.full((LANES,), 1, jnp.int32)
    zi = jnp.zeros((LANES,), jnp.int32)

    def zinit(j, _):
        hist_v[pl.ds(j * LANES, LANES)] = zi
        return ()

    lax.fori_loop(0, LANES * C_PAD // LANES, zinit, ())

    for s in range(SUB // TILE):
        pltpu.sync_copy(lab_hbm.at[w, pl.ds(s * TILE, TILE)], lab_t)

        def body(t, _):
            v_lab = lab_t[t]
            plsc.addupdate_scatter(hist_v, [lane_c + v_lab], ones)
            return ()

        lax.fori_loop(0, TILE, body, ())

    plsc.subcore_barrier()
    pl.delay(300)
    pltpu.sync_copy(hist_v, hist_hbm.at[pl.ds(w * LANES * C_PAD, LANES * C_PAD)])


def _prefix_kernel(hist_ref, base_ref, mass_ref):
    hist_f = hist_ref[...].astype(jnp.float32)
    r = lax.broadcasted_iota(jnp.int32, (ROWS, ROWS), 0)
    c = lax.broadcasted_iota(jnp.int32, (ROWS, ROWS), 1)
    ltri = (c < r).astype(jnp.float32)
    base_f = jnp.dot(ltri, hist_f, precision=lax.Precision.HIGHEST)
    base_ref[...] = base_f.astype(jnp.int32)
    totals = base_f[ROWS - 1:ROWS, :] + hist_f[ROWS - 1:ROWS, :]
    mass_f = jnp.floor(totals / float(N_BINS))
    col = lax.broadcasted_iota(jnp.int32, (1, C_PAD), 1)
    mass_ref[...] = jnp.where(col < NUM_CLASS, mass_f, 0.0).astype(jnp.int32)


def _binning_body(lab_hbm, conf_hbm, hit_hbm, base_hbm, mass_hbm,
                  acc_c_hbm, acc_h_hbm,
                  lab_t, conf_t, hit_t, cnt_v, mass_v, acc_cv, acc_hv):
    w = lax.axis_index("s") * 2 + lax.axis_index("c")
    lane = lax.iota(jnp.int32, LANES)
    lane_c = lane * C_PAD
    lane_a = lane * ACC_W
    zf = jnp.zeros((LANES,), jnp.float32)

    def zinit(j, _):
        acc_cv[pl.ds(j * LANES, LANES)] = zf
        acc_hv[pl.ds(j * LANES, LANES)] = zf
        return ()

    lax.fori_loop(0, LANES * ACC_W // LANES, zinit, ())

    pltpu.sync_copy(base_hbm.at[pl.ds(w * LANES * C_PAD, LANES * C_PAD)], cnt_v)
    pltpu.sync_copy(mass_hbm, mass_v)

    for s in range(SUB // TILE):
        pltpu.sync_copy(lab_hbm.at[w, pl.ds(s * TILE, TILE)], lab_t)
        pltpu.sync_copy(conf_hbm.at[w, pl.ds(s * TILE, TILE)], conf_t)
        pltpu.sync_copy(hit_hbm.at[w, pl.ds(s * TILE, TILE)], hit_t)

        def body(t, _):
            v_lab = lab_t[t]
            v_conf = conf_t[t]
            v_hit = hit_t[t]
            cidx = lane_c + v_lab
            rank = plsc.load_gather(cnt_v, [cidx])
            plsc.store_scatter(cnt_v, [cidx], rank + 1)
            m = plsc.load_gather(mass_v, [v_lab])
            msafe = jnp.maximum(m, 1)
            b = lax.div(rank, msafe)
            valid = (m > 0) & (rank < m * N_BINS)
            fidx = lane_a + v_lab * N_BINS + b
            plsc.addupdate_scatter(acc_cv, [fidx], v_conf, mask=valid)
            plsc.addupdate_scatter(acc_hv, [fidx], v_hit, mask=valid)
            return ()

        lax.fori_loop(0, TILE, body, ())

    plsc.subcore_barrier()
    pl.delay(300)
    pltpu.sync_copy(acc_cv, acc_c_hbm.at[pl.ds(w * LANES * ACC_W, LANES * ACC_W)])
    pltpu.sync_copy(acc_hv, acc_h_hbm.at[pl.ds(w * LANES * ACC_W, LANES * ACC_W)])


def _combine_kernel(acc_c_ref, acc_h_ref, mass_rep_ref, out_ref):
    cs = jnp.sum(acc_c_ref[...], axis=0, keepdims=True)
    hs = jnp.sum(acc_h_ref[...], axis=0, keepdims=True)
    mrep = mass_rep_ref[...]
    d = jnp.abs(cs / mrep - hs / mrep)
    f = lax.broadcasted_iota(jnp.int32, (ACC_W, NUM_CLASS), 0)
    c = lax.broadcasted_iota(jnp.int32, (ACC_W, NUM_CLASS), 1)
    sel = (lax.div(f, N_BINS) == c).astype(jnp.float32)
    ece = jnp.dot(d, sel, precision=lax.Precision.HIGHEST) / float(N_BINS)
    num = jnp.sum(ece * ece)
    den = jnp.sum(ece)
    out_ref[...] = (num / den).reshape(1, 1)


def kernel(confidences, hits, labels):
    n = labels.shape[0]
    pad = N_PADDED - n
    lab_p = jnp.concatenate(
        [labels.astype(jnp.int32),
         jnp.full((pad,), PAD_CLASS, jnp.int32)])
    conf_p = jnp.concatenate([confidences, jnp.zeros((pad,), jnp.float32)])
    hit_p = jnp.concatenate([hits, jnp.zeros((pad,), jnp.float32)])
    lab3 = lab_p.reshape(NW, LANES, SUB).transpose(0, 2, 1)
    conf3 = conf_p.reshape(NW, LANES, SUB).transpose(0, 2, 1)
    hit3 = hit_p.reshape(NW, LANES, SUB).transpose(0, 2, 1)

    mesh = plsc.VectorSubcoreMesh(core_axis_name="c", subcore_axis_name="s")
    sc_params = pltpu.CompilerParams(
        needs_layout_passes=False, use_tc_tiling_on_sc=False)

    hist1 = pl.kernel(
        _histogram_body,
        mesh=mesh,
        compiler_params=sc_params,
        out_type=jax.ShapeDtypeStruct((HIST_N,), jnp.int32),
        scratch_types=[
            pltpu.VMEM((TILE, LANES), jnp.int32),
            pltpu.VMEM((LANES * C_PAD,), jnp.int32),
        ],
    )(lab3)

    base2, mass2 = pl.pallas_call(
        _prefix_kernel,
        out_shape=(
            jax.ShapeDtypeStruct((ROWS, C_PAD), jnp.int32),
            jax.ShapeDtypeStruct((1, C_PAD), jnp.int32),
        ),
    )(hist1.reshape(ROWS, C_PAD))

    acc_c1, acc_h1 = pl.kernel(
        _binning_body,
        mesh=mesh,
        compiler_params=sc_params,
        out_type=(
            jax.ShapeDtypeStruct((ACC_N,), jnp.float32),
            jax.ShapeDtypeStruct((ACC_N,), jnp.float32),
        ),
        scratch_types=[
            pltpu.VMEM((TILE, LANES), jnp.int32),
            pltpu.VMEM((TILE, LANES), jnp.float32),
            pltpu.VMEM((TILE, LANES), jnp.float32),
            pltpu.VMEM((LANES * C_PAD,), jnp.int32),
            pltpu.VMEM((C_PAD,), jnp.int32),
            pltpu.VMEM((LANES * ACC_W,), jnp.float32),
            pltpu.VMEM((LANES * ACC_W,), jnp.float32),
        ],
    )(lab3, conf3, hit3, base2.reshape(HIST_N), mass2.reshape(C_PAD))

    mass_f = mass2.reshape(C_PAD)[:NUM_CLASS].astype(jnp.float32)
    mass_rep = jnp.concatenate(
        [jnp.repeat(mass_f, N_BINS),
         jnp.ones((ACC_W - NUM_CLASS * N_BINS,), jnp.float32)]
    ).reshape(1, ACC_W)

    out = pl.pallas_call(
        _combine_kernel,
        out_shape=jax.ShapeDtypeStruct((1, 1), jnp.float32),
    )(acc_c1.reshape(ROWS, ACC_W), acc_h1.reshape(ROWS, ACC_W), mass_rep)
    return out[0, 0]


# speedup vs baseline: 96.4615x; 1.4654x over previous
import jax
import jax.numpy as jnp
from jax import lax
from jax.experimental import pallas as pl
from jax.experimental.pallas import tpu as pltpu
from jax.experimental.pallas import tpu_sc as plsc

NUM_CLASS = 100
N_BINS = 15
PAD_CLASS = 100
C_PAD = 128
NW = 32
LANES = 16
SUB = 4096
TILE = 512
NSTAGE = SUB // TILE
N_PADDED = NW * LANES * SUB
ROWS = NW * LANES
ACC_W = 1504
HIST_N = ROWS * C_PAD
ACC_N = ROWS * ACC_W


def _histogram_body(lab_hbm, hist_hbm, lab_t, hist_v, sem):
    w = lax.axis_index("s") * 2 + lax.axis_index("c")
    lane = lax.iota(jnp.int32, LANES)
    lane_c = lane * C_PAD
    ones = jnp.full((LANES,), 1, jnp.int32)
    zi = jnp.zeros((LANES,), jnp.int32)

    cps = [None, None]
    cps[0] = pltpu.async_copy(
        lab_hbm.at[w, pl.ds(0, TILE)], lab_t.at[0], sem)

    def zinit(j, _):
        hist_v[pl.ds(j * LANES, LANES)] = zi
        return ()

    lax.fori_loop(0, LANES * C_PAD // LANES, zinit, ())

    for s in range(NSTAGE):
        cur = s % 2
        cps[cur].wait()
        if s + 1 < NSTAGE:
            cps[1 - cur] = pltpu.async_copy(
                lab_hbm.at[w, pl.ds((s + 1) * TILE, TILE)],
                lab_t.at[1 - cur], sem)

        def body(t, _):
            v0 = lab_t[cur, t * 2]
            plsc.addupdate_scatter(hist_v, [lane_c + v0], ones)
            v1 = lab_t[cur, t * 2 + 1]
            plsc.addupdate_scatter(hist_v, [lane_c + v1], ones)
            return ()

        lax.fori_loop(0, TILE // 2, body, ())

    plsc.subcore_barrier()
    pl.delay(300)
    pltpu.sync_copy(hist_v, hist_hbm.at[pl.ds(w * LANES * C_PAD, LANES * C_PAD)])


def _prefix_kernel(hist_ref, base_ref, mass_ref):
    hist_f = hist_ref[...].astype(jnp.float32)
    r = lax.broadcasted_iota(jnp.int32, (ROWS, ROWS), 0)
    c = lax.broadcasted_iota(jnp.int32, (ROWS, ROWS), 1)
    ltri = (c < r).astype(jnp.float32)
    base_f = jnp.dot(ltri, hist_f, precision=lax.Precision.HIGHEST)
    base_ref[...] = base_f.astype(jnp.int32)
    totals = base_f[ROWS - 1:ROWS, :] + hist_f[ROWS - 1:ROWS, :]
    mass_f = jnp.floor(totals / float(N_BINS))
    col = lax.broadcasted_iota(jnp.int32, (1, C_PAD), 1)
    mass_ref[...] = jnp.where(col < NUM_CLASS, mass_f, 0.0).astype(jnp.int32)


def _binning_body(lab_hbm, conf_hbm, hit_hbm, base_hbm, mass_hbm,
                  acc_c_hbm, acc_h_hbm,
                  lab_t, conf_t, hit_t, cnt_v, mass_v, acc_cv, acc_hv, sem):
    w = lax.axis_index("s") * 2 + lax.axis_index("c")
    lane = lax.iota(jnp.int32, LANES)
    lane_c = lane * C_PAD
    lane_a = lane * ACC_W
    zf = jnp.zeros((LANES,), jnp.float32)

    def issue(s, buf):
        return [
            pltpu.async_copy(
                lab_hbm.at[w, pl.ds(s * TILE, TILE)], lab_t.at[buf], sem),
            pltpu.async_copy(
                conf_hbm.at[w, pl.ds(s * TILE, TILE)], conf_t.at[buf], sem),
            pltpu.async_copy(
                hit_hbm.at[w, pl.ds(s * TILE, TILE)], hit_t.at[buf], sem),
        ]

    cps = [None, None]
    cps[0] = issue(0, 0)

    pltpu.sync_copy(base_hbm.at[pl.ds(w * LANES * C_PAD, LANES * C_PAD)], cnt_v)
    pltpu.sync_copy(mass_hbm, mass_v)

    def zinit(j, _):
        acc_cv[pl.ds(j * LANES, LANES)] = zf
        acc_hv[pl.ds(j * LANES, LANES)] = zf
        return ()

    lax.fori_loop(0, LANES * ACC_W // LANES, zinit, ())

    for s in range(NSTAGE):
        cur = s % 2
        for cp in cps[cur]:
            cp.wait()
        if s + 1 < NSTAGE:
            cps[1 - cur] = issue(s + 1, 1 - cur)

        def body(t, _):
            for u in range(2):
                tt = t * 2 + u
                v_lab = lab_t[cur, tt]
                v_conf = conf_t[cur, tt]
                v_hit = hit_t[cur, tt]
                cidx = lane_c + v_lab
                rank = plsc.load_gather(cnt_v, [cidx])
                plsc.store_scatter(cnt_v, [cidx], rank + 1)
                m = plsc.load_gather(mass_v, [v_lab])
                msafe = jnp.maximum(m, 1)
                b = (rank.astype(jnp.float32)
                     / msafe.astype(jnp.float32)).astype(jnp.int32)
                valid = (m > 0) & (rank < m * N_BINS)
                fidx = lane_a + v_lab * N_BINS + b
                plsc.addupdate_scatter(acc_cv, [fidx], v_conf, mask=valid)
                plsc.addupdate_scatter(acc_hv, [fidx], v_hit, mask=valid)
            return ()

        lax.fori_loop(0, TILE // 2, body, ())

    plsc.subcore_barrier()
    pl.delay(300)
    pltpu.sync_copy(acc_cv, acc_c_hbm.at[pl.ds(w * LANES * ACC_W, LANES * ACC_W)])
    pltpu.sync_copy(acc_hv, acc_h_hbm.at[pl.ds(w * LANES * ACC_W, LANES * ACC_W)])


def _combine_kernel(acc_c_ref, acc_h_ref, mass_rep_ref, out_ref):
    cs = jnp.sum(acc_c_ref[...], axis=0, keepdims=True)
    hs = jnp.sum(acc_h_ref[...], axis=0, keepdims=True)
    mrep = mass_rep_ref[...]
    d = jnp.abs(cs / mrep - hs / mrep)
    f = lax.broadcasted_iota(jnp.int32, (ACC_W, NUM_CLASS), 0)
    c = lax.broadcasted_iota(jnp.int32, (ACC_W, NUM_CLASS), 1)
    sel = (lax.div(f, N_BINS) == c).astype(jnp.float32)
    ece = jnp.dot(d, sel, precision=lax.Precision.HIGHEST) / float(N_BINS)
    num = jnp.sum(ece * ece)
    den = jnp.sum(ece)
    out_ref[...] = (num / den).reshape(1, 1)


def kernel(confidences, hits, labels):
    n = labels.shape[0]
    pad = N_PADDED - n
    lab_p = jnp.concatenate(
        [labels.astype(jnp.int32),
         jnp.full((pad,), PAD_CLASS, jnp.int32)])
    conf_p = jnp.concatenate([confidences, jnp.zeros((pad,), jnp.float32)])
    hit_p = jnp.concatenate([hits, jnp.zeros((pad,), jnp.float32)])
    lab3 = lab_p.reshape(NW, LANES, SUB).transpose(0, 2, 1)
    conf3 = conf_p.reshape(NW, LANES, SUB).transpose(0, 2, 1)
    hit3 = hit_p.reshape(NW, LANES, SUB).transpose(0, 2, 1)

    mesh = plsc.VectorSubcoreMesh(core_axis_name="c", subcore_axis_name="s")
    sc_params = pltpu.CompilerParams(
        needs_layout_passes=False, use_tc_tiling_on_sc=False)

    hist1 = pl.kernel(
        _histogram_body,
        mesh=mesh,
        compiler_params=sc_params,
        out_type=jax.ShapeDtypeStruct((HIST_N,), jnp.int32),
        scratch_types=[
            pltpu.VMEM((2, TILE, LANES), jnp.int32),
            pltpu.VMEM((LANES * C_PAD,), jnp.int32),
            pltpu.SemaphoreType.DMA,
        ],
    )(lab3)

    base2, mass2 = pl.pallas_call(
        _prefix_kernel,
        out_shape=(
            jax.ShapeDtypeStruct((ROWS, C_PAD), jnp.int32),
            jax.ShapeDtypeStruct((1, C_PAD), jnp.int32),
        ),
    )(hist1.reshape(ROWS, C_PAD))

    acc_c1, acc_h1 = pl.kernel(
        _binning_body,
        mesh=mesh,
        compiler_params=sc_params,
        out_type=(
            jax.ShapeDtypeStruct((ACC_N,), jnp.float32),
            jax.ShapeDtypeStruct((ACC_N,), jnp.float32),
        ),
        scratch_types=[
            pltpu.VMEM((2, TILE, LANES), jnp.int32),
            pltpu.VMEM((2, TILE, LANES), jnp.float32),
            pltpu.VMEM((2, TILE, LANES), jnp.float32),
            pltpu.VMEM((LANES * C_PAD,), jnp.int32),
            pltpu.VMEM((C_PAD,), jnp.int32),
            pltpu.VMEM((LANES * ACC_W,), jnp.float32),
            pltpu.VMEM((LANES * ACC_W,), jnp.float32),
            pltpu.SemaphoreType.DMA,
        ],
    )(lab3, conf3, hit3, base2.reshape(HIST_N), mass2.reshape(C_PAD))

    mass_f = mass2.reshape(C_PAD)[:NUM_CLASS].astype(jnp.float32)
    mass_rep = jnp.concatenate(
        [jnp.repeat(mass_f, N_BINS),
         jnp.ones((ACC_W - NUM_CLASS * N_BINS,), jnp.float32)]
    ).reshape(1, ACC_W)

    out = pl.pallas_call(
        _combine_kernel,
        out_shape=jax.ShapeDtypeStruct((1, 1), jnp.float32),
    )(acc_c1.reshape(ROWS, ACC_W), acc_h1.reshape(ROWS, ACC_W), mass_rep)
    return out[0, 0]


# speedup vs baseline: 96.8096x; 1.0036x over previous
import jax
import jax.numpy as jnp
from jax import lax
from jax.experimental import pallas as pl
from jax.experimental.pallas import tpu as pltpu
from jax.experimental.pallas import tpu_sc as plsc

NUM_CLASS = 100
N_BINS = 15
PAD_CLASS = 100
C_PAD = 128
NW = 32
LANES = 16
SUB = 4096
TILE = 512
NSTAGE = SUB // TILE
N_PADDED = NW * LANES * SUB
ROWS = NW * LANES
ACC_W = 1504
HIST_N = ROWS * C_PAD
ACC_N = ROWS * ACC_W


def _histogram_body(lab_hbm, hist_hbm, lab_t, hist_v, sem):
    w = lax.axis_index("s") * 2 + lax.axis_index("c")
    lane = lax.iota(jnp.int32, LANES)
    lane_c = lane * C_PAD
    ones = jnp.full((LANES,), 1, jnp.int32)
    zi = jnp.zeros((LANES,), jnp.int32)

    cps = [None, None]
    cps[0] = pltpu.async_copy(
        lab_hbm.at[w, pl.ds(0, TILE)], lab_t.at[0], sem)

    def zinit(j, _):
        hist_v[pl.ds(j * LANES, LANES)] = zi
        return ()

    lax.fori_loop(0, LANES * C_PAD // LANES, zinit, ())

    for s in range(NSTAGE):
        cur = s % 2
        cps[cur].wait()
        if s + 1 < NSTAGE:
            cps[1 - cur] = pltpu.async_copy(
                lab_hbm.at[w, pl.ds((s + 1) * TILE, TILE)],
                lab_t.at[1 - cur], sem)

        def body(t, _):
            for u in range(4):
                v = lab_t[cur, t * 4 + u]
                plsc.addupdate_scatter(hist_v, [lane_c + v], ones)
            return ()

        lax.fori_loop(0, TILE // 4, body, ())

    plsc.subcore_barrier()
    pl.delay(300)
    pltpu.sync_copy(hist_v, hist_hbm.at[pl.ds(w * LANES * C_PAD, LANES * C_PAD)])


def _prefix_kernel(hist_ref, base_ref, mass_ref):
    hist_f = hist_ref[...].astype(jnp.float32)
    r = lax.broadcasted_iota(jnp.int32, (ROWS, ROWS), 0)
    c = lax.broadcasted_iota(jnp.int32, (ROWS, ROWS), 1)
    ltri = (c < r).astype(jnp.float32)
    base_f = jnp.dot(ltri, hist_f, precision=lax.Precision.HIGHEST)
    base_ref[...] = base_f.astype(jnp.int32)
    totals = base_f[ROWS - 1:ROWS, :] + hist_f[ROWS - 1:ROWS, :]
    mass_f = jnp.floor(totals / float(N_BINS))
    col = lax.broadcasted_iota(jnp.int32, (1, C_PAD), 1)
    mass_ref[...] = jnp.where(col < NUM_CLASS, mass_f, 0.0).astype(jnp.int32)


def _binning_body(lab_hbm, conf_hbm, hit_hbm, base_hbm, mass_hbm,
                  acc_c_hbm, acc_h_hbm,
                  lab_t, conf_t, hit_t, cnt_v, mass_v, acc_cv, acc_hv, sem):
    w = lax.axis_index("s") * 2 + lax.axis_index("c")
    lane = lax.iota(jnp.int32, LANES)
    lane_c = lane * C_PAD
    lane_a = lane * ACC_W
    zf = jnp.zeros((LANES,), jnp.float32)

    def issue(s, buf):
        return [
            pltpu.async_copy(
                lab_hbm.at[w, pl.ds(s * TILE, TILE)], lab_t.at[buf], sem),
            pltpu.async_copy(
                conf_hbm.at[w, pl.ds(s * TILE, TILE)], conf_t.at[buf], sem),
            pltpu.async_copy(
                hit_hbm.at[w, pl.ds(s * TILE, TILE)], hit_t.at[buf], sem),
        ]

    cps = [None, None]
    cps[0] = issue(0, 0)

    pltpu.sync_copy(base_hbm.at[pl.ds(w * LANES * C_PAD, LANES * C_PAD)], cnt_v)
    pltpu.sync_copy(mass_hbm, mass_v)

    def zinit(j, _):
        acc_cv[pl.ds(j * LANES, LANES)] = zf
        acc_hv[pl.ds(j * LANES, LANES)] = zf
        return ()

    lax.fori_loop(0, LANES * ACC_W // LANES, zinit, ())

    for s in range(NSTAGE):
        cur = s % 2
        for cp in cps[cur]:
            cp.wait()
        if s + 1 < NSTAGE:
            cps[1 - cur] = issue(s + 1, 1 - cur)

        def body(t, _):
            for u in range(4):
                tt = t * 4 + u
                v_lab = lab_t[cur, tt]
                v_conf = conf_t[cur, tt]
                v_hit = hit_t[cur, tt]
                cidx = lane_c + v_lab
                rank = plsc.load_gather(cnt_v, [cidx])
                plsc.store_scatter(cnt_v, [cidx], rank + 1)
                m = plsc.load_gather(mass_v, [v_lab])
                msafe = jnp.maximum(m, 1)
                b = (rank.astype(jnp.float32)
                     / msafe.astype(jnp.float32)).astype(jnp.int32)
                valid = (m > 0) & (rank < m * N_BINS)
                fidx = lane_a + v_lab * N_BINS + b
                plsc.addupdate_scatter(acc_cv, [fidx], v_conf, mask=valid)
                plsc.addupdate_scatter(acc_hv, [fidx], v_hit, mask=valid)
            return ()

        lax.fori_loop(0, TILE // 4, body, ())

    plsc.subcore_barrier()
    pl.delay(300)
    pltpu.sync_copy(acc_cv, acc_c_hbm.at[pl.ds(w * LANES * ACC_W, LANES * ACC_W)])
    pltpu.sync_copy(acc_hv, acc_h_hbm.at[pl.ds(w * LANES * ACC_W, LANES * ACC_W)])


def _combine_kernel(acc_c_ref, acc_h_ref, mass_rep_ref, out_ref):
    cs = jnp.sum(acc_c_ref[...], axis=0, keepdims=True)
    hs = jnp.sum(acc_h_ref[...], axis=0, keepdims=True)
    mrep = mass_rep_ref[...]
    d = jnp.abs(cs / mrep - hs / mrep)
    f = lax.broadcasted_iota(jnp.int32, (ACC_W, NUM_CLASS), 0)
    c = lax.broadcasted_iota(jnp.int32, (ACC_W, NUM_CLASS), 1)
    sel = (lax.div(f, N_BINS) == c).astype(jnp.float32)
    ece = jnp.dot(d, sel, precision=lax.Precision.HIGHEST) / float(N_BINS)
    num = jnp.sum(ece * ece)
    den = jnp.sum(ece)
    out_ref[...] = (num / den).reshape(1, 1)


def kernel(confidences, hits, labels):
    n = labels.shape[0]
    pad = N_PADDED - n
    lab_p = jnp.concatenate(
        [labels.astype(jnp.int32),
         jnp.full((pad,), PAD_CLASS, jnp.int32)])
    conf_p = jnp.concatenate([confidences, jnp.zeros((pad,), jnp.float32)])
    hit_p = jnp.concatenate([hits, jnp.zeros((pad,), jnp.float32)])
    lab3 = lab_p.reshape(NW, LANES, SUB).transpose(0, 2, 1)
    conf3 = conf_p.reshape(NW, LANES, SUB).transpose(0, 2, 1)
    hit3 = hit_p.reshape(NW, LANES, SUB).transpose(0, 2, 1)

    mesh = plsc.VectorSubcoreMesh(core_axis_name="c", subcore_axis_name="s")
    sc_params = pltpu.CompilerParams(
        needs_layout_passes=False, use_tc_tiling_on_sc=False)

    hist1 = pl.kernel(
        _histogram_body,
        mesh=mesh,
        compiler_params=sc_params,
        out_type=jax.ShapeDtypeStruct((HIST_N,), jnp.int32),
        scratch_types=[
            pltpu.VMEM((2, TILE, LANES), jnp.int32),
            pltpu.VMEM((LANES * C_PAD,), jnp.int32),
            pltpu.SemaphoreType.DMA,
        ],
    )(lab3)

    base2, mass2 = pl.pallas_call(
        _prefix_kernel,
        out_shape=(
            jax.ShapeDtypeStruct((ROWS, C_PAD), jnp.int32),
            jax.ShapeDtypeStruct((1, C_PAD), jnp.int32),
        ),
    )(hist1.reshape(ROWS, C_PAD))

    acc_c1, acc_h1 = pl.kernel(
        _binning_body,
        mesh=mesh,
        compiler_params=sc_params,
        out_type=(
            jax.ShapeDtypeStruct((ACC_N,), jnp.float32),
            jax.ShapeDtypeStruct((ACC_N,), jnp.float32),
        ),
        scratch_types=[
            pltpu.VMEM((2, TILE, LANES), jnp.int32),
            pltpu.VMEM((2, TILE, LANES), jnp.float32),
            pltpu.VMEM((2, TILE, LANES), jnp.float32),
            pltpu.VMEM((LANES * C_PAD,), jnp.int32),
            pltpu.VMEM((C_PAD,), jnp.int32),
            pltpu.VMEM((LANES * ACC_W,), jnp.float32),
            pltpu.VMEM((LANES * ACC_W,), jnp.float32),
            pltpu.SemaphoreType.DMA,
        ],
    )(lab3, conf3, hit3, base2.reshape(HIST_N), mass2.reshape(C_PAD))

    mass_f = mass2.reshape(C_PAD)[:NUM_CLASS].astype(jnp.float32)
    mass_rep = jnp.concatenate(
        [jnp.repeat(mass_f, N_BINS),
         jnp.ones((ACC_W - NUM_CLASS * N_BINS,), jnp.float32)]
    ).reshape(1, ACC_W)

    out = pl.pallas_call(
        _combine_kernel,
        out_shape=jax.ShapeDtypeStruct((1, 1), jnp.float32),
    )(acc_c1.reshape(ROWS, ACC_W), acc_h1.reshape(ROWS, ACC_W), mass_rep)
    return out[0, 0]


# speedup vs baseline: 103.8736x; 1.0730x over previous
import jax
import jax.numpy as jnp
from jax import lax
from jax.experimental import pallas as pl
from jax.experimental.pallas import tpu as pltpu
from jax.experimental.pallas import tpu_sc as plsc

NUM_CLASS = 100
N_BINS = 15
PAD_CLASS = 100
C_PAD = 128
NW = 32
LANES = 16
SUB = 4096
TILE = 512
NSTAGE = SUB // TILE
N_PADDED = NW * LANES * SUB
ROWS = NW * LANES
ACC_W = 1504
HIST_N = ROWS * C_PAD
ACC_N = ROWS * ACC_W


def _histogram_body(lab_hbm, hist_hbm, lab_t, hist_v, sem):
    w = lax.axis_index("s") * 2 + lax.axis_index("c")
    lane = lax.iota(jnp.int32, LANES)
    lane_c = lane * C_PAD
    ones = jnp.full((LANES,), 1, jnp.int32)
    zi = jnp.zeros((LANES,), jnp.int32)

    cps = [None, None]
    cps[0] = pltpu.async_copy(
        lab_hbm.at[w, pl.ds(0, TILE)], lab_t.at[0], sem)

    def zinit(j, _):
        hist_v[pl.ds(j * LANES, LANES)] = zi
        return ()

    lax.fori_loop(0, LANES * C_PAD // LANES, zinit, ())

    for s in range(NSTAGE):
        cur = s % 2
        cps[cur].wait()
        if s + 1 < NSTAGE:
            cps[1 - cur] = pltpu.async_copy(
                lab_hbm.at[w, pl.ds((s + 1) * TILE, TILE)],
                lab_t.at[1 - cur], sem)

        def body(t, _):
            for u in range(4):
                v = lab_t[cur, t * 4 + u]
                plsc.addupdate_scatter(hist_v, [lane_c + v], ones)
            return ()

        lax.fori_loop(0, TILE // 4, body, ())

    plsc.subcore_barrier()
    pl.delay(300)
    pltpu.sync_copy(hist_v, hist_hbm.at[pl.ds(w * LANES * C_PAD, LANES * C_PAD)])


def _prefix_kernel(hist_ref, base_ref, mass_ref, rinv_ref):
    hist_f = hist_ref[...].astype(jnp.float32)
    r = lax.broadcasted_iota(jnp.int32, (ROWS, ROWS), 0)
    c = lax.broadcasted_iota(jnp.int32, (ROWS, ROWS), 1)
    ltri = (c < r).astype(jnp.float32)
    base_f = jnp.dot(ltri, hist_f, precision=lax.Precision.HIGHEST)
    base_ref[...] = base_f.astype(jnp.int32)
    totals = base_f[ROWS - 1:ROWS, :] + hist_f[ROWS - 1:ROWS, :]
    mass_f = jnp.floor(totals / float(N_BINS))
    col = lax.broadcasted_iota(jnp.int32, (1, C_PAD), 1)
    live = (col < NUM_CLASS) & (mass_f > 0)
    mass_ref[...] = jnp.where(live, mass_f, 0.0).astype(jnp.int32)
    rinv_ref[...] = jnp.where(live, 1.0 / jnp.maximum(mass_f, 1.0), 0.0)


def _binning_body(lab_hbm, conf_hbm, hit_hbm, base_hbm, mass_hbm,
                  acc_c_hbm, acc_h_hbm,
                  lab_t, conf_t, hit_t, cnt_v, mass_v, acc_cv, acc_hv, sem):
    w = lax.axis_index("s") * 2 + lax.axis_index("c")
    lane = lax.iota(jnp.int32, LANES)
    lane_c = lane * C_PAD
    lane_a = lane * ACC_W
    zf = jnp.zeros((LANES,), jnp.float32)

    def issue(s, buf):
        return [
            pltpu.async_copy(
                lab_hbm.at[w, pl.ds(s * TILE, TILE)], lab_t.at[buf], sem),
            pltpu.async_copy(
                conf_hbm.at[w, pl.ds(s * TILE, TILE)], conf_t.at[buf], sem),
            pltpu.async_copy(
                hit_hbm.at[w, pl.ds(s * TILE, TILE)], hit_t.at[buf], sem),
        ]

    cps = [None, None]
    cps[0] = issue(0, 0)

    pltpu.sync_copy(base_hbm.at[pl.ds(w * LANES * C_PAD, LANES * C_PAD)], cnt_v)
    pltpu.sync_copy(mass_hbm, mass_v)

    def zinit(j, _):
        acc_cv[pl.ds(j * LANES, LANES)] = zf
        acc_hv[pl.ds(j * LANES, LANES)] = zf
        return ()

    lax.fori_loop(0, LANES * ACC_W // LANES, zinit, ())

    for s in range(NSTAGE):
        cur = s % 2
        for cp in cps[cur]:
            cp.wait()
        if s + 1 < NSTAGE:
            cps[1 - cur] = issue(s + 1, 1 - cur)

        def body(t, _):
            for u in range(4):
                tt = t * 4 + u
                v_lab = lab_t[cur, tt]
                v_conf = conf_t[cur, tt]
                v_hit = hit_t[cur, tt]
                cidx = lane_c + v_lab
                rank = plsc.load_gather(cnt_v, [cidx])
                plsc.store_scatter(cnt_v, [cidx], rank + 1)
                rinv = plsc.load_gather(mass_v, [v_lab])
                bf = (rank.astype(jnp.float32) + 0.5) * rinv
                b = bf.astype(jnp.int32)
                valid = (rinv > 0.0) & (b < N_BINS)
                fidx = lane_a + v_lab * N_BINS + b
                plsc.addupdate_scatter(acc_cv, [fidx], v_conf, mask=valid)
                plsc.addupdate_scatter(acc_hv, [fidx], v_hit, mask=valid)
            return ()

        lax.fori_loop(0, TILE // 4, body, ())

    plsc.subcore_barrier()
    pl.delay(300)
    pltpu.sync_copy(acc_cv, acc_c_hbm.at[pl.ds(w * LANES * ACC_W, LANES * ACC_W)])
    pltpu.sync_copy(acc_hv, acc_h_hbm.at[pl.ds(w * LANES * ACC_W, LANES * ACC_W)])


def _combine_kernel(acc_c_ref, acc_h_ref, mass_rep_ref, out_ref):
    cs = jnp.sum(acc_c_ref[...], axis=0, keepdims=True)
    hs = jnp.sum(acc_h_ref[...], axis=0, keepdims=True)
    mrep = mass_rep_ref[...]
    d = jnp.abs(cs / mrep - hs / mrep)
    f = lax.broadcasted_iota(jnp.int32, (ACC_W, NUM_CLASS), 0)
    c = lax.broadcasted_iota(jnp.int32, (ACC_W, NUM_CLASS), 1)
    sel = (lax.div(f, N_BINS) == c).astype(jnp.float32)
    ece = jnp.dot(d, sel, precision=lax.Precision.HIGHEST) / float(N_BINS)
    num = jnp.sum(ece * ece)
    den = jnp.sum(ece)
    out_ref[...] = (num / den).reshape(1, 1)


def kernel(confidences, hits, labels):
    n = labels.shape[0]
    pad = N_PADDED - n
    lab_p = jnp.concatenate(
        [labels.astype(jnp.int32),
         jnp.full((pad,), PAD_CLASS, jnp.int32)])
    conf_p = jnp.concatenate([confidences, jnp.zeros((pad,), jnp.float32)])
    hit_p = jnp.concatenate([hits, jnp.zeros((pad,), jnp.float32)])
    lab3 = lab_p.reshape(NW, LANES, SUB).transpose(0, 2, 1)
    conf3 = conf_p.reshape(NW, LANES, SUB).transpose(0, 2, 1)
    hit3 = hit_p.reshape(NW, LANES, SUB).transpose(0, 2, 1)

    mesh = plsc.VectorSubcoreMesh(core_axis_name="c", subcore_axis_name="s")
    sc_params = pltpu.CompilerParams(
        needs_layout_passes=False, use_tc_tiling_on_sc=False)

    hist1 = pl.kernel(
        _histogram_body,
        mesh=mesh,
        compiler_params=sc_params,
        out_type=jax.ShapeDtypeStruct((HIST_N,), jnp.int32),
        scratch_types=[
            pltpu.VMEM((2, TILE, LANES), jnp.int32),
            pltpu.VMEM((LANES * C_PAD,), jnp.int32),
            pltpu.SemaphoreType.DMA,
        ],
    )(lab3)

    base2, mass2, rinv2 = pl.pallas_call(
        _prefix_kernel,
        out_shape=(
            jax.ShapeDtypeStruct((ROWS, C_PAD), jnp.int32),
            jax.ShapeDtypeStruct((1, C_PAD), jnp.int32),
            jax.ShapeDtypeStruct((1, C_PAD), jnp.float32),
        ),
    )(hist1.reshape(ROWS, C_PAD))

    acc_c1, acc_h1 = pl.kernel(
        _binning_body,
        mesh=mesh,
        compiler_params=sc_params,
        out_type=(
            jax.ShapeDtypeStruct((ACC_N,), jnp.float32),
            jax.ShapeDtypeStruct((ACC_N,), jnp.float32),
        ),
        scratch_types=[
            pltpu.VMEM((2, TILE, LANES), jnp.int32),
            pltpu.VMEM((2, TILE, LANES), jnp.float32),
            pltpu.VMEM((2, TILE, LANES), jnp.float32),
            pltpu.VMEM((LANES * C_PAD,), jnp.int32),
            pltpu.VMEM((C_PAD,), jnp.float32),
            pltpu.VMEM((LANES * ACC_W,), jnp.float32),
            pltpu.VMEM((LANES * ACC_W,), jnp.float32),
            pltpu.SemaphoreType.DMA,
        ],
    )(lab3, conf3, hit3, base2.reshape(HIST_N), rinv2.reshape(C_PAD))

    mass_f = mass2.reshape(C_PAD)[:NUM_CLASS].astype(jnp.float32)
    mass_rep = jnp.concatenate(
        [jnp.repeat(mass_f, N_BINS),
         jnp.ones((ACC_W - NUM_CLASS * N_BINS,), jnp.float32)]
    ).reshape(1, ACC_W)

    out = pl.pallas_call(
        _combine_kernel,
        out_shape=jax.ShapeDtypeStruct((1, 1), jnp.float32),
    )(acc_c1.reshape(ROWS, ACC_W), acc_h1.reshape(ROWS, ACC_W), mass_rep)
    return out[0, 0]


# speedup vs baseline: 104.5809x; 1.0068x over previous
import jax
import jax.numpy as jnp
from jax import lax
from jax.experimental import pallas as pl
from jax.experimental.pallas import tpu as pltpu
from jax.experimental.pallas import tpu_sc as plsc

NUM_CLASS = 100
N_BINS = 15
PAD_CLASS = 100
C_PAD = 128
NW = 32
LANES = 16
SUB = 4096
TILE = 512
NSTAGE = SUB // TILE
N_PADDED = NW * LANES * SUB
ROWS = NW * LANES
ACC_W = 1504
HIST_N = ROWS * C_PAD
ACC_N = ROWS * ACC_W


def _histogram_body(lab_hbm, hist_hbm, lab_t, hist_v, sem):
    w = lax.axis_index("s") * 2 + lax.axis_index("c")
    lane = lax.iota(jnp.int32, LANES)
    lane_c = lane * C_PAD
    ones = jnp.full((LANES,), 1, jnp.int32)
    zi = jnp.zeros((LANES,), jnp.int32)

    def issue(s, buf):
        return pltpu.async_copy(
            lab_hbm.at[pl.ds(w * LANES, LANES), pl.ds(s * TILE, TILE)],
            lab_t.at[buf], sem)

    cps = [None, None]
    cps[0] = issue(0, 0)

    def zinit(j, _):
        hist_v[pl.ds(j * LANES, LANES)] = zi
        return ()

    lax.fori_loop(0, LANES * C_PAD // LANES, zinit, ())

    for s in range(NSTAGE):
        cur = s % 2
        cps[cur].wait()
        if s + 1 < NSTAGE:
            cps[1 - cur] = issue(s + 1, 1 - cur)
        bufv = jnp.full((LANES,), cur, jnp.int32)

        def body(t, _):
            for u in range(4):
                ttv = jnp.full((LANES,), t * 4 + u, jnp.int32)
                v = plsc.load_gather(lab_t, [bufv, lane, ttv])
                plsc.addupdate_scatter(hist_v, [lane_c + v], ones)
            return ()

        lax.fori_loop(0, TILE // 4, body, ())

    plsc.subcore_barrier()
    pl.delay(300)
    pltpu.sync_copy(hist_v, hist_hbm.at[pl.ds(w * LANES * C_PAD, LANES * C_PAD)])


def _prefix_kernel(hist_ref, base_ref, mass_ref, rinv_ref):
    hist_f = hist_ref[...].astype(jnp.float32)
    r = lax.broadcasted_iota(jnp.int32, (ROWS, ROWS), 0)
    c = lax.broadcasted_iota(jnp.int32, (ROWS, ROWS), 1)
    ltri = (c < r).astype(jnp.float32)
    base_f = jnp.dot(ltri, hist_f, precision=lax.Precision.HIGHEST)
    base_ref[...] = base_f.astype(jnp.int32)
    totals = base_f[ROWS - 1:ROWS, :] + hist_f[ROWS - 1:ROWS, :]
    mass_f = jnp.floor(totals / float(N_BINS))
    col = lax.broadcasted_iota(jnp.int32, (1, C_PAD), 1)
    live = (col < NUM_CLASS) & (mass_f > 0)
    mass_ref[...] = jnp.where(live, mass_f, 0.0).astype(jnp.int32)
    rinv_ref[...] = jnp.where(live, 1.0 / jnp.maximum(mass_f, 1.0), 0.0)


def _binning_body(lab_hbm, conf_hbm, hit_hbm, base_hbm, mass_hbm,
                  acc_c_hbm, acc_h_hbm,
                  lab_t, conf_t, hit_t, cnt_v, mass_v, acc_cv, acc_hv, sem):
    w = lax.axis_index("s") * 2 + lax.axis_index("c")
    lane = lax.iota(jnp.int32, LANES)
    lane_c = lane * C_PAD
    lane_a = lane * ACC_W
    zf = jnp.zeros((LANES,), jnp.float32)

    def issue(s, buf):
        rows = pl.ds(w * LANES, LANES)
        cols = pl.ds(s * TILE, TILE)
        return [
            pltpu.async_copy(lab_hbm.at[rows, cols], lab_t.at[buf], sem),
            pltpu.async_copy(conf_hbm.at[rows, cols], conf_t.at[buf], sem),
            pltpu.async_copy(hit_hbm.at[rows, cols], hit_t.at[buf], sem),
        ]

    cps = [None, None]
    cps[0] = issue(0, 0)

    pltpu.sync_copy(base_hbm.at[pl.ds(w * LANES * C_PAD, LANES * C_PAD)], cnt_v)
    pltpu.sync_copy(mass_hbm, mass_v)

    def zinit(j, _):
        acc_cv[pl.ds(j * LANES, LANES)] = zf
        acc_hv[pl.ds(j * LANES, LANES)] = zf
        return ()

    lax.fori_loop(0, LANES * ACC_W // LANES, zinit, ())

    for s in range(NSTAGE):
        cur = s % 2
        for cp in cps[cur]:
            cp.wait()
        if s + 1 < NSTAGE:
            cps[1 - cur] = issue(s + 1, 1 - cur)
        bufv = jnp.full((LANES,), cur, jnp.int32)

        def body(t, _):
            for u in range(4):
                ttv = jnp.full((LANES,), t * 4 + u, jnp.int32)
                v_lab = plsc.load_gather(lab_t, [bufv, lane, ttv])
                v_conf = plsc.load_gather(conf_t, [bufv, lane, ttv])
                v_hit = plsc.load_gather(hit_t, [bufv, lane, ttv])
                cidx = lane_c + v_lab
                rank = plsc.load_gather(cnt_v, [cidx])
                plsc.store_scatter(cnt_v, [cidx], rank + 1)
                rinv = plsc.load_gather(mass_v, [v_lab])
                bf = (rank.astype(jnp.float32) + 0.5) * rinv
                b = bf.astype(jnp.int32)
                valid = (rinv > 0.0) & (b < N_BINS)
                fidx = lane_a + v_lab * N_BINS + b
                plsc.addupdate_scatter(acc_cv, [fidx], v_conf, mask=valid)
                plsc.addupdate_scatter(acc_hv, [fidx], v_hit, mask=valid)
            return ()

        lax.fori_loop(0, TILE // 4, body, ())

    plsc.subcore_barrier()
    pl.delay(300)
    pltpu.sync_copy(acc_cv, acc_c_hbm.at[pl.ds(w * LANES * ACC_W, LANES * ACC_W)])
    pltpu.sync_copy(acc_hv, acc_h_hbm.at[pl.ds(w * LANES * ACC_W, LANES * ACC_W)])


def _combine_kernel(acc_c_ref, acc_h_ref, mass_rep_ref, out_ref):
    cs = jnp.sum(acc_c_ref[...], axis=0, keepdims=True)
    hs = jnp.sum(acc_h_ref[...], axis=0, keepdims=True)
    mrep = mass_rep_ref[...]
    d = jnp.abs(cs / mrep - hs / mrep)
    f = lax.broadcasted_iota(jnp.int32, (ACC_W, NUM_CLASS), 0)
    c = lax.broadcasted_iota(jnp.int32, (ACC_W, NUM_CLASS), 1)
    sel = (lax.div(f, N_BINS) == c).astype(jnp.float32)
    ece = jnp.dot(d, sel, precision=lax.Precision.HIGHEST) / float(N_BINS)
    num = jnp.sum(ece * ece)
    den = jnp.sum(ece)
    out_ref[...] = (num / den).reshape(1, 1)


def kernel(confidences, hits, labels):
    n = labels.shape[0]
    pad = N_PADDED - n
    lab_p = jnp.concatenate(
        [labels.astype(jnp.int32),
         jnp.full((pad,), PAD_CLASS, jnp.int32)])
    conf_p = jnp.concatenate([confidences, jnp.zeros((pad,), jnp.float32)])
    hit_p = jnp.concatenate([hits, jnp.zeros((pad,), jnp.float32)])
    lab3 = lab_p.reshape(ROWS, SUB)
    conf3 = conf_p.reshape(ROWS, SUB)
    hit3 = hit_p.reshape(ROWS, SUB)

    mesh = plsc.VectorSubcoreMesh(core_axis_name="c", subcore_axis_name="s")
    sc_params = pltpu.CompilerParams(
        needs_layout_passes=False, use_tc_tiling_on_sc=False)

    hist1 = pl.kernel(
        _histogram_body,
        mesh=mesh,
        compiler_params=sc_params,
        out_type=jax.ShapeDtypeStruct((HIST_N,), jnp.int32),
        scratch_types=[
            pltpu.VMEM((2, LANES, TILE), jnp.int32),
            pltpu.VMEM((LANES * C_PAD,), jnp.int32),
            pltpu.SemaphoreType.DMA,
        ],
    )(lab3)

    base2, mass2, rinv2 = pl.pallas_call(
        _prefix_kernel,
        out_shape=(
            jax.ShapeDtypeStruct((ROWS, C_PAD), jnp.int32),
            jax.ShapeDtypeStruct((1, C_PAD), jnp.int32),
            jax.ShapeDtypeStruct((1, C_PAD), jnp.float32),
        ),
    )(hist1.reshape(ROWS, C_PAD))

    acc_c1, acc_h1 = pl.kernel(
        _binning_body,
        mesh=mesh,
        compiler_params=sc_params,
        out_type=(
            jax.ShapeDtypeStruct((ACC_N,), jnp.float32),
            jax.ShapeDtypeStruct((ACC_N,), jnp.float32),
        ),
        scratch_types=[
            pltpu.VMEM((2, LANES, TILE), jnp.int32),
            pltpu.VMEM((2, LANES, TILE), jnp.float32),
            pltpu.VMEM((2, LANES, TILE), jnp.float32),
            pltpu.VMEM((LANES * C_PAD,), jnp.int32),
            pltpu.VMEM((C_PAD,), jnp.float32),
            pltpu.VMEM((LANES * ACC_W,), jnp.float32),
            pltpu.VMEM((LANES * ACC_W,), jnp.float32),
            pltpu.SemaphoreType.DMA,
        ],
    )(lab3, conf3, hit3, base2.reshape(HIST_N), rinv2.reshape(C_PAD))

    mass_f = mass2.reshape(C_PAD)[:NUM_CLASS].astype(jnp.float32)
    mass_rep = jnp.concatenate(
        [jnp.repeat(mass_f, N_BINS),
         jnp.ones((ACC_W - NUM_CLASS * N_BINS,), jnp.float32)]
    ).reshape(1, ACC_W)

    out = pl.pallas_call(
        _combine_kernel,
        out_shape=jax.ShapeDtypeStruct((1, 1), jnp.float32),
    )(acc_c1.reshape(ROWS, ACC_W), acc_h1.reshape(ROWS, ACC_W), mass_rep)
    return out[0, 0]


# speedup vs baseline: 138.9322x; 1.3285x over previous
import jax
import jax.numpy as jnp
from jax import lax
from jax.experimental import pallas as pl
from jax.experimental.pallas import tpu as pltpu
from jax.experimental.pallas import tpu_sc as plsc

NUM_CLASS = 100
N_BINS = 15
PAD_CLASS = 100
C_PAD = 128
NW = 32
LANES = 16
SUB = 4096
TILE = 512
NSTAGE = SUB // TILE
N_PADDED = NW * LANES * SUB
ROWS = NW * LANES
ACC_W = 1504
HIST_N = ROWS * C_PAD
ACC_N = ROWS * ACC_W


def _histogram_body(lab_hbm, hist_hbm, lab_t, hist_v, sem):
    w = lax.axis_index("s") * 2 + lax.axis_index("c")
    lane = lax.iota(jnp.int32, LANES)
    lane_c = lane * C_PAD
    ones = jnp.full((LANES,), 1, jnp.int32)
    zi = jnp.zeros((LANES,), jnp.int32)

    def issue(s, buf):
        return pltpu.async_copy(
            lab_hbm.at[pl.ds(w * LANES, LANES), pl.ds(s * TILE, TILE)],
            lab_t.at[buf, :, pl.ds(0, TILE)], sem)

    cps = [None, None]
    cps[0] = issue(0, 0)

    def zinit(j, _):
        hist_v[pl.ds(j * LANES, LANES)] = zi
        return ()

    lax.fori_loop(0, LANES * C_PAD // LANES, zinit, ())

    for s in range(NSTAGE):
        cur = s % 2
        cps[cur].wait()
        if s + 1 < NSTAGE:
            cps[1 - cur] = issue(s + 1, 1 - cur)
        bufv = jnp.full((LANES,), cur, jnp.int32)

        def body(t, _):
            for u in range(4):
                ttv = jnp.full((LANES,), t * 4 + u, jnp.int32)
                v = plsc.load_gather(lab_t, [bufv, lane, ttv])
                plsc.addupdate_scatter(hist_v, [lane_c + v], ones)
            return ()

        lax.fori_loop(0, TILE // 4, body, ())

    plsc.subcore_barrier()
    pl.delay(300)
    pltpu.sync_copy(hist_v, hist_hbm.at[pl.ds(w * LANES * C_PAD, LANES * C_PAD)])


def _prefix_kernel(hist_ref, base_ref, mass_ref, rinv_ref):
    hist_f = hist_ref[...].astype(jnp.float32)
    r = lax.broadcasted_iota(jnp.int32, (ROWS, ROWS), 0)
    c = lax.broadcasted_iota(jnp.int32, (ROWS, ROWS), 1)
    ltri = (c < r).astype(jnp.float32)
    base_f = jnp.dot(ltri, hist_f, precision=lax.Precision.HIGHEST)
    base_ref[...] = base_f.astype(jnp.int32)
    totals = base_f[ROWS - 1:ROWS, :] + hist_f[ROWS - 1:ROWS, :]
    mass_f = jnp.floor(totals / float(N_BINS))
    col = lax.broadcasted_iota(jnp.int32, (1, C_PAD), 1)
    live = (col < NUM_CLASS) & (mass_f > 0)
    mass_ref[...] = jnp.where(live, mass_f, 0.0).astype(jnp.int32)
    rinv_ref[...] = jnp.where(live, 1.0 / jnp.maximum(mass_f, 1.0), 0.0)


def _binning_body(lab_hbm, conf_hbm, hit_hbm, base_hbm, mass_hbm,
                  acc_c_hbm, acc_h_hbm,
                  lab_t, conf_t, hit_t, cnt_v, mass_v, acc_cv, acc_hv, sem):
    w = lax.axis_index("s") * 2 + lax.axis_index("c")
    lane = lax.iota(jnp.int32, LANES)
    lane_c = lane * C_PAD
    lane_a = lane * ACC_W
    zf = jnp.zeros((LANES,), jnp.float32)

    def issue(s, buf):
        rows = pl.ds(w * LANES, LANES)
        cols = pl.ds(s * TILE, TILE)
        dst = (buf, slice(None), pl.ds(0, TILE))
        return [
            pltpu.async_copy(lab_hbm.at[rows, cols], lab_t.at[dst], sem),
            pltpu.async_copy(conf_hbm.at[rows, cols], conf_t.at[dst], sem),
            pltpu.async_copy(hit_hbm.at[rows, cols], hit_t.at[dst], sem),
        ]

    cps = [None, None]
    cps[0] = issue(0, 0)

    pltpu.sync_copy(base_hbm.at[pl.ds(w * LANES * C_PAD, LANES * C_PAD)], cnt_v)
    pltpu.sync_copy(mass_hbm, mass_v)

    def zinit(j, _):
        acc_cv[pl.ds(j * LANES, LANES)] = zf
        acc_hv[pl.ds(j * LANES, LANES)] = zf
        return ()

    lax.fori_loop(0, LANES * ACC_W // LANES, zinit, ())

    for s in range(NSTAGE):
        cur = s % 2
        for cp in cps[cur]:
            cp.wait()
        if s + 1 < NSTAGE:
            cps[1 - cur] = issue(s + 1, 1 - cur)
        bufv = jnp.full((LANES,), cur, jnp.int32)

        def body(t, _):
            for u in range(4):
                ttv = jnp.full((LANES,), t * 4 + u, jnp.int32)
                v_lab = plsc.load_gather(lab_t, [bufv, lane, ttv])
                v_conf = plsc.load_gather(conf_t, [bufv, lane, ttv])
                v_hit = plsc.load_gather(hit_t, [bufv, lane, ttv])
                cidx = lane_c + v_lab
                rank = plsc.load_gather(cnt_v, [cidx])
                plsc.store_scatter(cnt_v, [cidx], rank + 1)
                rinv = plsc.load_gather(mass_v, [v_lab])
                bf = (rank.astype(jnp.float32) + 0.5) * rinv
                b = bf.astype(jnp.int32)
                valid = (rinv > 0.0) & (b < N_BINS)
                fidx = lane_a + v_lab * N_BINS + b
                plsc.addupdate_scatter(acc_cv, [fidx], v_conf, mask=valid)
                plsc.addupdate_scatter(acc_hv, [fidx], v_hit, mask=valid)
            return ()

        lax.fori_loop(0, TILE // 4, body, ())

    plsc.subcore_barrier()
    pl.delay(300)
    pltpu.sync_copy(acc_cv, acc_c_hbm.at[pl.ds(w * LANES * ACC_W, LANES * ACC_W)])
    pltpu.sync_copy(acc_hv, acc_h_hbm.at[pl.ds(w * LANES * ACC_W, LANES * ACC_W)])


def _combine_kernel(acc_c_ref, acc_h_ref, mass_rep_ref, out_ref):
    cs = jnp.sum(acc_c_ref[...], axis=0, keepdims=True)
    hs = jnp.sum(acc_h_ref[...], axis=0, keepdims=True)
    mrep = mass_rep_ref[...]
    d = jnp.abs(cs / mrep - hs / mrep)
    f = lax.broadcasted_iota(jnp.int32, (ACC_W, NUM_CLASS), 0)
    c = lax.broadcasted_iota(jnp.int32, (ACC_W, NUM_CLASS), 1)
    sel = (lax.div(f, N_BINS) == c).astype(jnp.float32)
    ece = jnp.dot(d, sel, precision=lax.Precision.HIGHEST) / float(N_BINS)
    num = jnp.sum(ece * ece)
    den = jnp.sum(ece)
    out_ref[...] = (num / den).reshape(1, 1)


def kernel(confidences, hits, labels):
    n = labels.shape[0]
    pad = N_PADDED - n
    lab_p = jnp.concatenate(
        [labels.astype(jnp.int32),
         jnp.full((pad,), PAD_CLASS, jnp.int32)])
    conf_p = jnp.concatenate([confidences, jnp.zeros((pad,), jnp.float32)])
    hit_p = jnp.concatenate([hits, jnp.zeros((pad,), jnp.float32)])
    lab3 = lab_p.reshape(ROWS, SUB)
    conf3 = conf_p.reshape(ROWS, SUB)
    hit3 = hit_p.reshape(ROWS, SUB)

    mesh = plsc.VectorSubcoreMesh(core_axis_name="c", subcore_axis_name="s")
    sc_params = pltpu.CompilerParams(
        needs_layout_passes=False, use_tc_tiling_on_sc=False)

    hist1 = pl.kernel(
        _histogram_body,
        mesh=mesh,
        compiler_params=sc_params,
        out_type=jax.ShapeDtypeStruct((HIST_N,), jnp.int32),
        scratch_types=[
            pltpu.VMEM((2, LANES, TILE + 1), jnp.int32),
            pltpu.VMEM((LANES * C_PAD,), jnp.int32),
            pltpu.SemaphoreType.DMA,
        ],
    )(lab3)

    base2, mass2, rinv2 = pl.pallas_call(
        _prefix_kernel,
        out_shape=(
            jax.ShapeDtypeStruct((ROWS, C_PAD), jnp.int32),
            jax.ShapeDtypeStruct((1, C_PAD), jnp.int32),
            jax.ShapeDtypeStruct((1, C_PAD), jnp.float32),
        ),
    )(hist1.reshape(ROWS, C_PAD))

    acc_c1, acc_h1 = pl.kernel(
        _binning_body,
        mesh=mesh,
        compiler_params=sc_params,
        out_type=(
            jax.ShapeDtypeStruct((ACC_N,), jnp.float32),
            jax.ShapeDtypeStruct((ACC_N,), jnp.float32),
        ),
        scratch_types=[
            pltpu.VMEM((2, LANES, TILE + 1), jnp.int32),
            pltpu.VMEM((2, LANES, TILE + 1), jnp.float32),
            pltpu.VMEM((2, LANES, TILE + 1), jnp.float32),
            pltpu.VMEM((LANES * C_PAD,), jnp.int32),
            pltpu.VMEM((C_PAD,), jnp.float32),
            pltpu.VMEM((LANES * ACC_W,), jnp.float32),
            pltpu.VMEM((LANES * ACC_W,), jnp.float32),
            pltpu.SemaphoreType.DMA,
        ],
    )(lab3, conf3, hit3, base2.reshape(HIST_N), rinv2.reshape(C_PAD))

    mass_f = mass2.reshape(C_PAD)[:NUM_CLASS].astype(jnp.float32)
    mass_rep = jnp.concatenate(
        [jnp.repeat(mass_f, N_BINS),
         jnp.ones((ACC_W - NUM_CLASS * N_BINS,), jnp.float32)]
    ).reshape(1, ACC_W)

    out = pl.pallas_call(
        _combine_kernel,
        out_shape=jax.ShapeDtypeStruct((1, 1), jnp.float32),
    )(acc_c1.reshape(ROWS, ACC_W), acc_h1.reshape(ROWS, ACC_W), mass_rep)
    return out[0, 0]
